# baseline (device time: 83076 ns/iter reference)
import jax
import jax.numpy as jnp
from jax import lax
from jax.experimental import pallas as pl
from jax.experimental.pallas import tpu as pltpu

N_DEV = 4
B, SQ, HQ, HKV, DH = 4, 256, 8, 2, 128
GQ = HQ // HKV
SKV_LOC = 1024
D = HQ * DH
R = GQ * SQ
NBG = B * HKV
SCALE = 0.08838834764831843


def kernel(x, Wq, Wo, K_ext, V_ext):
    xb = x.reshape(B * SQ, D).astype(jnp.bfloat16)
    Wqb = Wq.astype(jnp.bfloat16)
    Wob = Wo.astype(jnp.bfloat16)
    Kb = K_ext.astype(jnp.bfloat16).transpose(0, 2, 1, 3)
    Vb = V_ext.astype(jnp.bfloat16).transpose(0, 2, 1, 3)
    Vb = jnp.concatenate([Vb, jnp.ones_like(Vb)], axis=-1)

    def body(x_ref, wq_ref, wo_ref, k_ref, v_ref, out_ref,
             gbuf, qs, send_sems, recv_sems):
        my = lax.axis_index("i")
        left = (my + N_DEV - 1) % N_DEV
        right = (my + 1) % N_DEV

        bsem = pltpu.get_barrier_semaphore()
        for nbr in (left, right):
            pl.semaphore_signal(
                bsem, inc=1, device_id=(nbr,),
                device_id_type=pl.DeviceIdType.MESH,
            )
        pl.semaphore_wait(bsem, 2)

        q = jnp.dot(x_ref[...], wq_ref[...], preferred_element_type=jnp.float32)
        q = (q * SCALE).astype(jnp.bfloat16)
        for b in range(B):
            for g in range(HKV):
                qs[b * HKV + g] = jnp.concatenate(
                    [q[b * SQ:(b + 1) * SQ, (g * GQ + t) * DH:(g * GQ + t + 1) * DH]
                     for t in range(GQ)],
                    axis=0,
                )

        def one(j):
            b = j // HKV
            g = j % HKV
            s = lax.dot_general(
                qs[j], k_ref[b, g], (((1,), (1,)), ((), ())),
                preferred_element_type=jnp.float32,
            )
            p = jnp.exp(s).astype(jnp.bfloat16)
            pv = jnp.dot(
                p, v_ref[b, g], preferred_element_type=jnp.float32,
            )
            gbuf[0, j] = pv[:, :DH].astype(jnp.bfloat16)
            gbuf[0, NBG, :, j:j + 1] = pv[:, DH:DH + 1].astype(jnp.bfloat16)

        HALves = (pl.ds(0, 4), pl.ds(4, NBG + 1 - 4))

        def xfer(src_slot, dst_slot, half, target):
            return pltpu.make_async_remote_copy(
                src_ref=gbuf.at[src_slot, HALves[half]],
                dst_ref=gbuf.at[dst_slot, HALves[half]],
                send_sem=send_sems.at[dst_slot, half],
                recv_sem=recv_sems.at[dst_slot, half],
                device_id=(target,), device_id_type=pl.DeviceIdType.MESH,
            )

        for j in range(4):
            one(j)
        s1a_r = xfer(0, 1, 0, right)
        s1a_l = xfer(0, 2, 0, left)
        s1a_r.start()
        s1a_l.start()

        for j in range(4, NBG):
            one(j)
        s1b_r = xfer(0, 1, 1, right)
        s1b_l = xfer(0, 2, 1, left)
        s1b_r.start()
        s1b_l.start()

        xfer(1, 1, 0, right).wait_recv()
        relay_r = xfer(1, 3, 0, right)
        relay_r.start()
        xfer(2, 2, 1, right).wait_recv()
        relay_l = xfer(2, 3, 1, left)
        relay_l.start()
        xfer(1, 1, 1, right).wait_recv()
        xfer(2, 2, 0, right).wait_recv()

        def l_col(slot, j):
            return gbuf[slot, NBG, :, j:j + 1].astype(jnp.float32)

        acc3 = []
        l3 = []
        for j in range(NBG):
            acc3.append(
                gbuf[0, j].astype(jnp.float32)
                + gbuf[1, j].astype(jnp.float32)
                + gbuf[2, j].astype(jnp.float32)
            )
            l3.append(l_col(0, j) + l_col(1, j) + l_col(2, j))

        xfer(3, 3, 0, right).wait_recv()
        xfer(3, 3, 1, right).wait_recv()

        o_rows = []
        for b in range(B):
            blocks = []
            for hh in range(HQ):
                g, t = hh // GQ, hh % GQ
                j = b * HKV + g
                rs = slice(t * SQ, (t + 1) * SQ)
                num = acc3[j][rs, :] + gbuf[3, j][rs, :].astype(jnp.float32)
                den = l3[j][rs, :] + l_col(3, j)[rs, :]
                blocks.append(num / den)
            o_rows.append(jnp.concatenate(blocks, axis=1))
        o = jnp.concatenate(o_rows, axis=0).astype(jnp.bfloat16)
        out = jnp.dot(o, wo_ref[...], preferred_element_type=jnp.float32)
        out_ref[...] = out.reshape(B, SQ, D)

        for d in (s1a_r, s1a_l, s1b_r, s1b_l, relay_r, relay_l):
            d.wait_send()

    return pl.pallas_call(
        body,
        out_shape=jax.ShapeDtypeStruct((B, SQ, D), jnp.float32),
        in_specs=[pl.BlockSpec(memory_space=pltpu.VMEM)] * 5,
        out_specs=pl.BlockSpec(memory_space=pltpu.VMEM),
        scratch_shapes=[
            pltpu.VMEM((N_DEV, NBG + 1, R, DH), jnp.bfloat16),
            pltpu.VMEM((NBG, R, DH), jnp.bfloat16),
            pltpu.SemaphoreType.DMA((N_DEV, 2)),
            pltpu.SemaphoreType.DMA((N_DEV, 2)),
        ],
        compiler_params=pltpu.CompilerParams(
            collective_id=0,
            vmem_limit_bytes=100 * 1024 * 1024,
        ),
    )(xb, Wqb, Wob, Kb, Vb)


# device time: 72141 ns/iter; 1.1516x vs baseline; 1.1516x over previous
import jax
import jax.numpy as jnp
from jax import lax
from jax.experimental import pallas as pl
from jax.experimental.pallas import tpu as pltpu

N_DEV = 4
B, SQ, HQ, HKV, DH = 4, 256, 8, 2, 128
GQ = HQ // HKV
SKV_LOC = 1024
D = HQ * DH
R = GQ * SQ
NBG = B * HKV
SCALE = 0.08838834764831843


def kernel(x, Wq, Wo, K_ext, V_ext):

    def body(x_ref, wq_ref, wo_ref, k_ref, v_ref, out_ref,
             gbuf, qs, send_sems, recv_sems):
        my = lax.axis_index("i")
        left = (my + N_DEV - 1) % N_DEV
        right = (my + 1) % N_DEV

        bsem = pltpu.get_barrier_semaphore()
        for nbr in (left, right):
            pl.semaphore_signal(
                bsem, inc=1, device_id=(nbr,),
                device_id_type=pl.DeviceIdType.MESH,
            )
        pl.semaphore_wait(bsem, 2)

        q = jnp.dot(
            x_ref[...].reshape(B * SQ, D).astype(jnp.bfloat16),
            wq_ref[...].astype(jnp.bfloat16),
            preferred_element_type=jnp.float32,
        )
        q = (q * SCALE).astype(jnp.bfloat16)
        for b in range(B):
            for g in range(HKV):
                qs[b * HKV + g] = jnp.concatenate(
                    [q[b * SQ:(b + 1) * SQ, (g * GQ + t) * DH:(g * GQ + t + 1) * DH]
                     for t in range(GQ)],
                    axis=0,
                )

        def one(j):
            b = j // HKV
            g = j % HKV
            kk = k_ref[b, :, g, :].astype(jnp.bfloat16)
            vv = v_ref[b, :, g, :].astype(jnp.bfloat16)
            s = lax.dot_general(
                qs[j], kk, (((1,), (1,)), ((), ())),
                preferred_element_type=jnp.float32,
            )
            p = jnp.exp(s)
            lj = jnp.sum(p, axis=-1, keepdims=True)
            gbuf[0, j] = jnp.dot(
                p.astype(jnp.bfloat16), vv, preferred_element_type=jnp.float32,
            ).astype(jnp.bfloat16)
            gbuf[0, NBG, :, j:j + 1] = lj.astype(jnp.bfloat16)

        HALves = (pl.ds(0, 4), pl.ds(4, NBG + 1 - 4))

        def xfer(src_slot, dst_slot, half, target):
            return pltpu.make_async_remote_copy(
                src_ref=gbuf.at[src_slot, HALves[half]],
                dst_ref=gbuf.at[dst_slot, HALves[half]],
                send_sem=send_sems.at[dst_slot, half],
                recv_sem=recv_sems.at[dst_slot, half],
                device_id=(target,), device_id_type=pl.DeviceIdType.MESH,
            )

        for j in range(4):
            one(j)
        s1a_r = xfer(0, 1, 0, right)
        s1a_l = xfer(0, 2, 0, left)
        s1a_r.start()
        s1a_l.start()

        for j in range(4, NBG):
            one(j)
        s1b_r = xfer(0, 1, 1, right)
        s1b_l = xfer(0, 2, 1, left)
        s1b_r.start()
        s1b_l.start()

        xfer(1, 1, 0, right).wait_recv()
        relay_r = xfer(1, 3, 0, right)
        relay_r.start()
        xfer(2, 2, 1, right).wait_recv()
        relay_l = xfer(2, 3, 1, left)
        relay_l.start()
        xfer(1, 1, 1, right).wait_recv()
        xfer(2, 2, 0, right).wait_recv()

        def l_col(slot, j):
            return gbuf[slot, NBG, :, j:j + 1].astype(jnp.float32)

        acc3 = []
        l3 = []
        for j in range(NBG):
            acc3.append(
                gbuf[0, j].astype(jnp.float32)
                + gbuf[1, j].astype(jnp.float32)
                + gbuf[2, j].astype(jnp.float32)
            )
            l3.append(l_col(0, j) + l_col(1, j) + l_col(2, j))

        xfer(3, 3, 0, right).wait_recv()
        xfer(3, 3, 1, right).wait_recv()

        o_rows = []
        for b in range(B):
            blocks = []
            for hh in range(HQ):
                g, t = hh // GQ, hh % GQ
                j = b * HKV + g
                rs = slice(t * SQ, (t + 1) * SQ)
                num = acc3[j][rs, :] + gbuf[3, j][rs, :].astype(jnp.float32)
                den = l3[j][rs, :] + l_col(3, j)[rs, :]
                blocks.append(num / den)
            o_rows.append(jnp.concatenate(blocks, axis=1))
        o = jnp.concatenate(o_rows, axis=0).astype(jnp.bfloat16)
        out = jnp.dot(o, wo_ref[...].astype(jnp.bfloat16),
                      preferred_element_type=jnp.float32)
        out_ref[...] = out.reshape(B, SQ, D)

        for d in (s1a_r, s1a_l, s1b_r, s1b_l, relay_r, relay_l):
            d.wait_send()

    return pl.pallas_call(
        body,
        out_shape=jax.ShapeDtypeStruct((B, SQ, D), jnp.float32),
        in_specs=[pl.BlockSpec(memory_space=pltpu.VMEM)] * 5,
        out_specs=pl.BlockSpec(memory_space=pltpu.VMEM),
        scratch_shapes=[
            pltpu.VMEM((N_DEV, NBG + 1, R, DH), jnp.bfloat16),
            pltpu.VMEM((NBG, R, DH), jnp.bfloat16),
            pltpu.SemaphoreType.DMA((N_DEV, 2)),
            pltpu.SemaphoreType.DMA((N_DEV, 2)),
        ],
        compiler_params=pltpu.CompilerParams(
            collective_id=0,
            vmem_limit_bytes=100 * 1024 * 1024,
        ),
    )(x, Wq, Wo, K_ext, V_ext)


# device time: 66343 ns/iter; 1.2522x vs baseline; 1.0874x over previous
import jax
import jax.numpy as jnp
from jax import lax
from jax.experimental import pallas as pl
from jax.experimental.pallas import tpu as pltpu

N_DEV = 4
B, SQ, HQ, HKV, DH = 4, 256, 8, 2, 128
GQ = HQ // HKV
SKV_LOC = 1024
D = HQ * DH
R = GQ * SQ
NBG = B * HKV
SCALE = 0.08838834764831843

L = NBG


def kernel(x, Wq, Wo, K_ext, V_ext):

    def body(x_ref, wq_ref, wo_ref, k_ref, v_ref, out_ref,
             gbuf, send_sems, recv_sems):
        my = lax.axis_index("i")
        left = (my + N_DEV - 1) % N_DEV
        right = (my + 1) % N_DEV

        bsem = pltpu.get_barrier_semaphore()
        for nbr in (left, right):
            pl.semaphore_signal(
                bsem, inc=1, device_id=(nbr,),
                device_id_type=pl.DeviceIdType.MESH,
            )
        pl.semaphore_wait(bsem, 2)

        def quarter_xfer(src_slot, dst_slot, q, target):
            return pltpu.make_async_remote_copy(
                src_ref=gbuf.at[src_slot, pl.ds(2 * q, 2)],
                dst_ref=gbuf.at[dst_slot, pl.ds(2 * q, 2)],
                send_sem=send_sems.at[dst_slot, q],
                recv_sem=recv_sems.at[dst_slot, q],
                device_id=(target,), device_id_type=pl.DeviceIdType.MESH,
            )

        def l_xfer(src_slot, dst_slot, target):
            return pltpu.make_async_remote_copy(
                src_ref=gbuf.at[src_slot, L],
                dst_ref=gbuf.at[dst_slot, L],
                send_sem=send_sems.at[dst_slot, 4],
                recv_sem=recv_sems.at[dst_slot, 4],
                device_id=(target,), device_id_type=pl.DeviceIdType.MESH,
            )

        def recvd(slot, q):
            d = quarter_xfer(slot, slot, q, right) if q < 4 else \
                l_xfer(slot, slot, right)
            d.wait_recv()

        wqb = wq_ref[...].astype(jnp.bfloat16)

        def one(j, qj):
            b = j // HKV
            g = j % HKV
            kk = k_ref[b, :, g, :].astype(jnp.bfloat16)
            vv = v_ref[b, :, g, :].astype(jnp.bfloat16)
            s = lax.dot_general(
                qj, kk, (((1,), (1,)), ((), ())),
                preferred_element_type=jnp.float32,
            )
            p = jnp.exp(s)
            lj = jnp.sum(p, axis=-1, keepdims=True)
            gbuf[0, j] = jnp.dot(
                p.astype(jnp.bfloat16), vv, preferred_element_type=jnp.float32,
            ).astype(jnp.bfloat16)
            gbuf[0, L, :, j:j + 1] = lj.astype(jnp.bfloat16)

        sends = []
        for b in range(B):
            q_b = jnp.dot(
                x_ref[b].astype(jnp.bfloat16), wqb,
                preferred_element_type=jnp.float32,
            )
            q_b = (q_b * SCALE).astype(jnp.bfloat16)
            for g in range(HKV):
                qj = jnp.concatenate(
                    [q_b[:, (g * GQ + t) * DH:(g * GQ + t + 1) * DH]
                     for t in range(GQ)],
                    axis=0,
                )
                one(b * HKV + g, qj)
            for dst_slot, target in ((1, right), (2, left)):
                d = quarter_xfer(0, dst_slot, b, target)
                d.start()
                sends.append(d)
        for dst_slot, target in ((1, right), (2, left)):
            d = l_xfer(0, dst_slot, target)
            d.start()
            sends.append(d)

        for q in (0, 1):
            recvd(1, q)
            d = quarter_xfer(1, 3, q, right)
            d.start()
            sends.append(d)
        for q in (2, 3):
            recvd(2, q)
            d = quarter_xfer(2, 3, q, left)
            d.start()
            sends.append(d)
        recvd(1, 4)
        d = l_xfer(1, 3, right)
        d.start()
        sends.append(d)

        acc = [None] * NBG
        for b in range(B):
            if b < 2:
                recvd(2, b)
            else:
                recvd(1, b)
            recvd(3, b)
            for j in (2 * b, 2 * b + 1):
                acc[j] = (
                    gbuf[0, j].astype(jnp.float32)
                    + gbuf[1, j].astype(jnp.float32)
                    + gbuf[2, j].astype(jnp.float32)
                    + gbuf[3, j].astype(jnp.float32)
                )

        recvd(2, 4)
        recvd(3, 4)
        wob = wo_ref[...].astype(jnp.bfloat16)

        def l_col(slot, j):
            return gbuf[slot, L, :, j:j + 1].astype(jnp.float32)

        for b in range(B):
            blocks = []
            for hh in range(HQ):
                g, t = hh // GQ, hh % GQ
                j = b * HKV + g
                rs = slice(t * SQ, (t + 1) * SQ)
                den = (l_col(0, j) + l_col(1, j) + l_col(2, j) + l_col(3, j))
                blocks.append(acc[j][rs, :] / den[rs, :])
            o_b = jnp.concatenate(blocks, axis=1).astype(jnp.bfloat16)
            out_ref[b] = jnp.dot(o_b, wob, preferred_element_type=jnp.float32)

        for d in sends:
            d.wait_send()

    return pl.pallas_call(
        body,
        out_shape=jax.ShapeDtypeStruct((B, SQ, D), jnp.float32),
        in_specs=[pl.BlockSpec(memory_space=pltpu.VMEM)] * 5,
        out_specs=pl.BlockSpec(memory_space=pltpu.VMEM),
        scratch_shapes=[
            pltpu.VMEM((N_DEV, NBG + 1, R, DH), jnp.bfloat16),
            pltpu.SemaphoreType.DMA((N_DEV, 5)),
            pltpu.SemaphoreType.DMA((N_DEV, 5)),
        ],
        compiler_params=pltpu.CompilerParams(
            collective_id=0,
            vmem_limit_bytes=100 * 1024 * 1024,
        ),
    )(x, Wq, Wo, K_ext, V_ext)


# device time: 64099 ns/iter; 1.2961x vs baseline; 1.0350x over previous
import jax
import jax.numpy as jnp
from jax import lax
from jax.experimental import pallas as pl
from jax.experimental.pallas import tpu as pltpu

N_DEV = 4
B, SQ, HQ, HKV, DH = 4, 256, 8, 2, 128
GQ = HQ // HKV
SKV_LOC = 1024
D = HQ * DH
R = GQ * SQ
SCALE = 0.08838834764831843

QROWS = 2 * R + 16


def kernel(x, Wq, Wo, K_ext, V_ext):

    def body(x_ref, wq_ref, wo_ref, k_ref, v_ref, out_ref,
             gbuf, send_sems, recv_sems):
        my = lax.axis_index("i")
        left = (my + N_DEV - 1) % N_DEV
        right = (my + 1) % N_DEV

        bsem = pltpu.get_barrier_semaphore()
        for nbr in (left, right):
            pl.semaphore_signal(
                bsem, inc=1, device_id=(nbr,),
                device_id_type=pl.DeviceIdType.MESH,
            )
        pl.semaphore_wait(bsem, 2)

        def xfer(src_slot, dst_slot, q, target):
            return pltpu.make_async_remote_copy(
                src_ref=gbuf.at[src_slot, q],
                dst_ref=gbuf.at[dst_slot, q],
                send_sem=send_sems.at[dst_slot, q],
                recv_sem=recv_sems.at[dst_slot, q],
                device_id=(target,), device_id_type=pl.DeviceIdType.MESH,
            )

        def recvd(slot, q):
            xfer(slot, slot, q, right).wait_recv()

        wqb = wq_ref[...].astype(jnp.bfloat16)
        sends = []

        for b in range(B):
            q_b = jnp.dot(
                x_ref[b].astype(jnp.bfloat16), wqb,
                preferred_element_type=jnp.float32,
            )
            q_b = (q_b * SCALE).astype(jnp.bfloat16)
            for g in range(HKV):
                qj = jnp.concatenate(
                    [q_b[:, (g * GQ + t) * DH:(g * GQ + t + 1) * DH]
                     for t in range(GQ)],
                    axis=0,
                )
                kk = k_ref[b, :, g, :].astype(jnp.bfloat16)
                vv = v_ref[b, :, g, :].astype(jnp.bfloat16)
                s = lax.dot_general(
                    qj, kk, (((1,), (1,)), ((), ())),
                    preferred_element_type=jnp.float32,
                )
                p = jnp.exp(s)
                lj = jnp.sum(p, axis=-1, keepdims=True)
                gbuf[0, b, g * R:(g + 1) * R, :] = jnp.dot(
                    p.astype(jnp.bfloat16), vv,
                    preferred_element_type=jnp.float32,
                ).astype(jnp.bfloat16)
                gbuf[0, b, 2 * R + 8 * g:2 * R + 8 * (g + 1), :] = (
                    lj.astype(jnp.bfloat16).reshape(8, DH))
            for dst_slot, target in ((1, right), (2, left)):
                d = xfer(0, dst_slot, b, target)
                d.start()
                sends.append(d)

        def relay(q):
            if q < 2:
                recvd(1, q)
                d = xfer(1, 3, q, right)
            else:
                recvd(2, q)
                d = xfer(2, 3, q, left)
            d.start()
            sends.append(d)

        wob = wo_ref[...].astype(jnp.bfloat16)

        def finalize(b):
            recvd(2 if b < 2 else 1, b)
            recvd(3, b)
            qa = (gbuf[0, b].astype(jnp.float32)
                  + gbuf[1, b].astype(jnp.float32)
                  + gbuf[2, b].astype(jnp.float32)
                  + gbuf[3, b].astype(jnp.float32))
            den16 = qa[2 * R:2 * R + 16, :]
            o3 = qa[:2 * R, :].reshape(16, 128, DH)
            o2 = (o3 / den16[:, :, None]).reshape(2 * R, DH)
            blocks = []
            for hh in range(HQ):
                g, t = hh // GQ, hh % GQ
                r0 = g * R + t * SQ
                blocks.append(o2[r0:r0 + SQ, :])
            o_b = jnp.concatenate(blocks, axis=1).astype(jnp.bfloat16)
            out_ref[b] = jnp.dot(o_b, wob, preferred_element_type=jnp.float32)

        relay(0)
        relay(1)
        relay(2)
        finalize(0)
        finalize(1)
        relay(3)
        finalize(2)
        finalize(3)

        for d in sends:
            d.wait_send()

    return pl.pallas_call(
        body,
        out_shape=jax.ShapeDtypeStruct((B, SQ, D), jnp.float32),
        in_specs=[pl.BlockSpec(memory_space=pltpu.VMEM)] * 5,
        out_specs=pl.BlockSpec(memory_space=pltpu.VMEM),
        scratch_shapes=[
            pltpu.VMEM((N_DEV, B, QROWS, DH), jnp.bfloat16),
            pltpu.SemaphoreType.DMA((N_DEV, B)),
            pltpu.SemaphoreType.DMA((N_DEV, B)),
        ],
        compiler_params=pltpu.CompilerParams(
            collective_id=0,
            vmem_limit_bytes=100 * 1024 * 1024,
        ),
    )(x, Wq, Wo, K_ext, V_ext)


# device time: 59165 ns/iter; 1.4041x vs baseline; 1.0834x over previous
import jax
import jax.numpy as jnp
from jax import lax
from jax.experimental import pallas as pl
from jax.experimental.pallas import tpu as pltpu

N_DEV = 4
B, SQ, HQ, HKV, DH = 4, 256, 8, 2, 128
GQ = HQ // HKV
SKV_LOC = 1024
D = HQ * DH
R = GQ * SQ
NBG = B * HKV
SCALE = 0.08838834764831843

BROWS = R + 8


def kernel(x, Wq, Wo, K_ext, V_ext):

    def body(x_ref, wq_ref, wo_ref, k_ref, v_ref, out_ref,
             gbuf, send_sems, recv_sems):
        my = lax.axis_index("i")
        left = (my + N_DEV - 1) % N_DEV
        right = (my + 1) % N_DEV

        bsem = pltpu.get_barrier_semaphore()
        for nbr in (left, right):
            pl.semaphore_signal(
                bsem, inc=1, device_id=(nbr,),
                device_id_type=pl.DeviceIdType.MESH,
            )
        pl.semaphore_wait(bsem, 2)

        def xfer(src_slot, dst_slot, j, target):
            return pltpu.make_async_remote_copy(
                src_ref=gbuf.at[src_slot, j],
                dst_ref=gbuf.at[dst_slot, j],
                send_sem=send_sems.at[dst_slot, j],
                recv_sem=recv_sems.at[dst_slot, j],
                device_id=(target,), device_id_type=pl.DeviceIdType.MESH,
            )

        def recvd(slot, j):
            xfer(slot, slot, j, right).wait_recv()

        wqb = wq_ref[...].astype(jnp.bfloat16)
        sends = []

        for b in range(B):
            q_b = jnp.dot(
                x_ref[b].astype(jnp.bfloat16), wqb,
                preferred_element_type=jnp.float32,
            )
            q_b = (q_b * SCALE).astype(jnp.bfloat16)
            for g in range(HKV):
                j = b * HKV + g
                qj = jnp.concatenate(
                    [q_b[:, (g * GQ + t) * DH:(g * GQ + t + 1) * DH]
                     for t in range(GQ)],
                    axis=0,
                )
                kk = k_ref[b, :, g, :].astype(jnp.bfloat16)
                vv = v_ref[b, :, g, :].astype(jnp.bfloat16)
                s = lax.dot_general(
                    qj, kk, (((1,), (1,)), ((), ())),
                    preferred_element_type=jnp.float32,
                )
                p = jnp.exp(s)
                lj = jnp.sum(p, axis=-1, keepdims=True)
                gbuf[0, j, :R, :] = jnp.dot(
                    p.astype(jnp.bfloat16), vv,
                    preferred_element_type=jnp.float32,
                ).astype(jnp.bfloat16)
                gbuf[0, j, R:, :] = lj.astype(jnp.bfloat16).reshape(8, DH)
                for dst_slot, target in ((1, right), (2, left)):
                    d = xfer(0, dst_slot, j, target)
                    d.start()
                    sends.append(d)

        def relay(j):
            if j < 4:
                recvd(1, j)
                d = xfer(1, 3, j, right)
            else:
                recvd(2, j)
                d = xfer(2, 3, j, left)
            d.start()
            sends.append(d)

        wob = wo_ref[...].astype(jnp.bfloat16)

        def premerge(j):
            recvd(2 if j < 4 else 1, j)
            return (gbuf[0, j].astype(jnp.float32)
                    + gbuf[1, j].astype(jnp.float32)
                    + gbuf[2, j].astype(jnp.float32))

        def complete(b, qa):
            o2s = []
            for g in range(HKV):
                j = b * HKV + g
                recvd(3, j)
                q4 = qa[j] + gbuf[3, j].astype(jnp.float32)
                den8 = q4[R:, :]
                o3 = q4[:R, :].reshape(8, 128, DH)
                o2s.append((o3 / den8[:, :, None]).reshape(R, DH))
            blocks = []
            for hh in range(HQ):
                g, t = hh // GQ, hh % GQ
                blocks.append(o2s[g][t * SQ:(t + 1) * SQ, :])
            o_b = jnp.concatenate(blocks, axis=1).astype(jnp.bfloat16)
            out_ref[b] = jnp.dot(o_b, wob, preferred_element_type=jnp.float32)

        qa = [None] * NBG
        for j in (0, 1, 2, 3):
            relay(j)
        for j in (4, 5):
            relay(j)
        for j in (0, 1, 2, 3):
            qa[j] = premerge(j)
        for j in (6, 7):
            relay(j)
        for j in (4, 5, 6, 7):
            qa[j] = premerge(j)
        for b in range(B):
            complete(b, qa)

        for d in sends:
            d.wait_send()

    return pl.pallas_call(
        body,
        out_shape=jax.ShapeDtypeStruct((B, SQ, D), jnp.float32),
        in_specs=[pl.BlockSpec(memory_space=pltpu.VMEM)] * 5,
        out_specs=pl.BlockSpec(memory_space=pltpu.VMEM),
        scratch_shapes=[
            pltpu.VMEM((N_DEV, NBG, BROWS, DH), jnp.bfloat16),
            pltpu.SemaphoreType.DMA((N_DEV, NBG)),
            pltpu.SemaphoreType.DMA((N_DEV, NBG)),
        ],
        compiler_params=pltpu.CompilerParams(
            collective_id=0,
            vmem_limit_bytes=100 * 1024 * 1024,
        ),
    )(x, Wq, Wo, K_ext, V_ext)
